# Initial kernel scaffold; baseline (speedup 1.0000x reference)
import jax
import jax.numpy as jnp
from jax.experimental import pallas as pl


def kernel(hidden_states, sim_matrix, gates, fallback_k):
    raise NotImplementedError("write your pallas kernel here")


# speedup vs baseline: 4.4485x; 4.4485x over previous
import jax
import jax.numpy as jnp
from jax.experimental import pallas as pl


_ROWS_PER_BLOCK = 2048


def _gating_block(x_ref, w_ref, g_ref, k_ref, rw_ref, lg_ref, am_ref):
    x = x_ref[...]
    w = w_ref[...]
    g = g_ref[...]
    kf = k_ref[...]

    w_norm = jnp.sqrt(jnp.sum(w * w, axis=0, keepdims=True))
    wn = w / jnp.maximum(w_norm, 1e-12)

    x_norm = jnp.sqrt(jnp.sum(x * x, axis=1, keepdims=True))
    xn = x / jnp.maximum(x_norm, 1e-12)

    logits = jnp.dot(xn, wn, preferred_element_type=jnp.float32) - g
    gated = jnp.maximum(logits, 0.0)
    act_mask = (logits > 0.0).astype(jnp.float32)
    inactive = jnp.max(logits, axis=1, keepdims=True) <= 0.0

    n_experts = logits.shape[1]
    cols = []
    for e in range(n_experts):
        le = logits[:, e : e + 1]
        gt = (logits > le).astype(jnp.float32)
        if e > 0:
            eq = (logits[:, :e] == le).astype(jnp.float32)
            rank_e = jnp.sum(gt, axis=1, keepdims=True) + jnp.sum(
                eq, axis=1, keepdims=True
            )
        else:
            rank_e = jnp.sum(gt, axis=1, keepdims=True)
        cols.append(rank_e)
    rank = jnp.concatenate(cols, axis=1)
    fb_mask = (rank < kf).astype(jnp.float32)

    mask = jnp.where(inactive, fb_mask, act_mask)
    neg = jnp.float32(-1e30)
    gated_masked = jnp.where(mask > 0.0, gated, neg)
    m = jnp.max(gated_masked, axis=1, keepdims=True)
    ex = jnp.exp(gated_masked - m)
    rw = ex / jnp.sum(ex, axis=1, keepdims=True)

    rw_ref[...] = rw
    lg_ref[...] = logits
    am_ref[...] = mask


def kernel(hidden_states, sim_matrix, gates, fallback_k):
    b, t, c = hidden_states.shape
    n = b * t
    e = sim_matrix.shape[1]
    flat = hidden_states.reshape(n, c)
    g2 = gates.reshape(1, e).astype(jnp.float32)
    kvec = jnp.full((1, e), fallback_k, jnp.float32)

    blk = _ROWS_PER_BLOCK
    grid = (n // blk,)
    out_shape = [jax.ShapeDtypeStruct((n, e), jnp.float32)] * 3
    rw, lg, am = pl.pallas_call(
        _gating_block,
        grid=grid,
        in_specs=[
            pl.BlockSpec((blk, c), lambda i: (i, 0)),
            pl.BlockSpec((c, e), lambda i: (0, 0)),
            pl.BlockSpec((1, e), lambda i: (0, 0)),
            pl.BlockSpec((1, e), lambda i: (0, 0)),
        ],
        out_specs=[pl.BlockSpec((blk, e), lambda i: (i, 0))] * 3,
        out_shape=out_shape,
    )(flat, sim_matrix, g2, kvec)
    return rw, lg, am
